# Initial kernel scaffold; baseline (speedup 1.0000x reference)
import jax
import jax.numpy as jnp
from jax.experimental import pallas as pl


def kernel(seq_len, past_len, embedding):
    raise NotImplementedError("write your pallas kernel here")


# speedup vs baseline: 1.4904x; 1.4904x over previous
import functools

import jax
import jax.numpy as jnp
from jax import lax
from jax.experimental import pallas as pl
from jax.experimental.pallas import tpu as pltpu
from jax.experimental.pallas import tpu_sc as plsc

MAX_ROWS = 8192
D = 2048
S = 4096

NC = 2
NS = 16
NW = NC * NS
ROWS_W = S // NW
CH = 8
CR = ROWS_W // CH


def _body(idx_hbm, table_hbm, out_hbm, idx_v, buf0, buf1, sem_g, sem_s):
    wid = lax.axis_index("s") * NC + lax.axis_index("c")
    base = wid * ROWS_W
    pltpu.sync_copy(idx_hbm.at[wid], idx_v)
    bufs = (buf0, buf1)

    def gather(c):
        return pltpu.async_copy(table_hbm.at[idx_v.at[c]], bufs[c % 2], sem_g)

    def scatter(c):
        return pltpu.async_copy(
            bufs[c % 2], out_hbm.at[pl.ds(base + c * CR, CR)], sem_s)

    scat = [None] * CH
    g = gather(0)
    for c in range(CH):
        g.wait()
        scat[c] = scatter(c)
        if c + 1 < CH:
            if c >= 1:
                scat[c - 1].wait()
            g = gather(c + 1)
    scat[CH - 2].wait()
    scat[CH - 1].wait()


@functools.partial(jax.jit)
def _sc_gather(idx, table):
    kern = functools.partial(
        pl.kernel,
        out_type=jax.ShapeDtypeStruct((S, D), jnp.float32),
        mesh=plsc.VectorSubcoreMesh(core_axis_name="c", subcore_axis_name="s"),
        scratch_types=[
            pltpu.VMEM((CH, 16), jnp.int32),
            pltpu.VMEM((CR, D), jnp.float32),
            pltpu.VMEM((CR, D), jnp.float32),
            pltpu.SemaphoreType.DMA,
            pltpu.SemaphoreType.DMA,
        ],
    )(_body)
    return kern(idx, table)


def kernel(seq_len, past_len, embedding):
    start = (jnp.asarray(past_len, jnp.int32)
             + jnp.asarray(seq_len, jnp.int32) - S)
    idx = (start + jnp.arange(S, dtype=jnp.int32)).reshape(NW, CH, 16)
    out = _sc_gather(idx, embedding)
    return out[None]
